# Initial kernel scaffold; baseline (speedup 1.0000x reference)
import jax
import jax.numpy as jnp
from jax.experimental import pallas as pl


def kernel(x, Wq, bq, Wk, bk, Wv, bv, Wa, ba, Wb, bb, Wo, bo):
    raise NotImplementedError("write your pallas kernel here")


# speedup vs baseline: 12.9352x; 12.9352x over previous
import functools

import jax
import jax.numpy as jnp
from jax.experimental import pallas as pl
from jax.experimental.pallas import tpu as pltpu

EPS = 1e-6
L = 128
HIGH = jax.lax.Precision.HIGHEST


def _dot(a, b, dims):
    return jax.lax.dot_general(a, b, (dims, ((), ())), precision=HIGH,
                               preferred_element_type=jnp.float32)


def _mm(a, b):
    return _dot(a, b, ((1,), (0,)))


def _mm_t(a, b):
    return _dot(a, b, ((1,), (1,)))


def _chunk_kernel(x_ref, wq, bq, wk, bk, wv, bv, wa, ba, wb, bb, wo, bo,
                  out_ref, S):
    j = pl.program_id(1)

    @pl.when(j == 0)
    def _():
        S[:] = jnp.zeros_like(S)

    xc = x_ref[0]

    def proj(w, bias):
        return _mm(xc, w[:]) + bias[0]

    def silu(t):
        return t * jax.nn.sigmoid(t)

    def l2n(t):
        return t / (jnp.sqrt(jnp.sum(t * t, axis=-1, keepdims=True)) + EPS)

    qc = l2n(silu(proj(wq, bq)))
    kc = l2n(silu(proj(wk, bk)))
    vc = silu(proj(wv, bv))
    ag = jnp.mean(jax.nn.sigmoid(proj(wa, ba)), axis=-1, keepdims=True)
    bg = jnp.mean(jax.nn.sigmoid(proj(wb, bb)), axis=-1, keepdims=True)

    row = jax.lax.broadcasted_iota(jnp.int32, (L, L), 0)
    col = jax.lax.broadcasted_iota(jnp.int32, (L, L), 1)

    la = jnp.log(jnp.maximum(ag, 1e-30))
    Lc = _mm((row >= col).astype(jnp.float32), la)
    A = jnp.exp(Lc)
    D = Lc - Lc.reshape(1, L)
    G_strict = jnp.exp(jnp.where(row > col, D, -1e30))
    G_incl = jnp.exp(jnp.where(row >= col, D, -1e30))
    eyeL = (row == col).astype(jnp.float32)

    St = S[:]
    KS0 = _mm_t(kc, St)
    RHS = bg * (vc - A * KS0)
    Nm = bg * (G_strict * _mm_t(kc, kc))
    P = eyeL - Nm
    Npow = Nm
    for _ in range(L.bit_length() - 2):
        Npow = _mm(Npow, Npow)
        P = _mm(P, eyeL + Npow)
    U = _mm(P, RHS)

    Pm = G_incl * _mm_t(qc, kc)
    O = A * _mm_t(qc, St) + _mm(Pm, U)
    out_ref[0] = _mm(O, wo[:]) + bo[0]

    lcl = Lc[L - 1, 0]
    gam = jnp.exp(lcl - Lc)
    S[:] = jnp.exp(lcl) * St + _dot(U * gam, kc, ((0,), (0,)))


@jax.jit
def kernel(x, Wq, bq, Wk, bk, Wv, bv, Wa, ba, Wb, bb, Wo, bo):
    B, N, C = x.shape
    grid = (B, N // L)
    wspec = pl.BlockSpec((C, C), lambda b, j: (0, 0))
    bspec = pl.BlockSpec((1, C), lambda b, j: (0, 0))
    xspec = pl.BlockSpec((1, L, C), lambda b, j: (b, j, 0))
    ws = [Wq.T, bq.reshape(1, C), Wk.T, bk.reshape(1, C), Wv.T,
          bv.reshape(1, C), Wa.T, ba.reshape(1, C), Wb.T, bb.reshape(1, C),
          Wo.T, bo.reshape(1, C)]
    out = pl.pallas_call(
        _chunk_kernel,
        grid=grid,
        in_specs=[xspec] + [wspec, bspec] * 6,
        out_specs=xspec,
        out_shape=jax.ShapeDtypeStruct((B, N, C), jnp.float32),
        scratch_shapes=[pltpu.VMEM((C, C), jnp.float32)],
        compiler_params=pltpu.CompilerParams(
            dimension_semantics=("parallel", "arbitrary")),
    )(x, *ws)
    return out


# speedup vs baseline: 25.3619x; 1.9607x over previous
import functools

import jax
import jax.numpy as jnp
from jax.experimental import pallas as pl
from jax.experimental.pallas import tpu as pltpu

EPS = 1e-6
L = 128
HIGH = jax.lax.Precision.DEFAULT


def _dot(a, b, dims):
    return jax.lax.dot_general(a, b, (dims, ((), ())), precision=HIGH,
                               preferred_element_type=jnp.float32)


def _mm(a, b):
    return _dot(a, b, ((1,), (0,)))


def _mm_t(a, b):
    return _dot(a, b, ((1,), (1,)))


def _chunk_kernel(x_ref, wq, bq, wk, bk, wv, bv, wa, ba, wb, bb, wo, bo,
                  out_ref, S):
    j = pl.program_id(1)

    @pl.when(j == 0)
    def _():
        S[:] = jnp.zeros_like(S)

    xc = x_ref[0]

    def proj(w, bias):
        return _mm(xc, w[:]) + bias[0]

    def silu(t):
        return t * jax.nn.sigmoid(t)

    def l2n(t):
        return t / (jnp.sqrt(jnp.sum(t * t, axis=-1, keepdims=True)) + EPS)

    qc = l2n(silu(proj(wq, bq)))
    kc = l2n(silu(proj(wk, bk)))
    vc = silu(proj(wv, bv))
    ag = jnp.mean(jax.nn.sigmoid(proj(wa, ba)), axis=-1, keepdims=True)
    bg = jnp.mean(jax.nn.sigmoid(proj(wb, bb)), axis=-1, keepdims=True)

    row = jax.lax.broadcasted_iota(jnp.int32, (L, L), 0)
    col = jax.lax.broadcasted_iota(jnp.int32, (L, L), 1)

    la = jnp.log(jnp.maximum(ag, 1e-30))
    Lc = _mm((row >= col).astype(jnp.float32), la)
    A = jnp.exp(Lc)
    D = Lc - Lc.reshape(1, L)
    G_strict = jnp.exp(jnp.where(row > col, D, -1e30))
    G_incl = jnp.exp(jnp.where(row >= col, D, -1e30))
    eyeL = (row == col).astype(jnp.float32)

    St = S[:]
    KS0 = _mm_t(kc, St)
    RHS = bg * (vc - A * KS0)
    Nm = bg * (G_strict * _mm_t(kc, kc))
    P = eyeL - Nm
    Npow = Nm
    for _ in range(L.bit_length() - 2):
        Npow = _mm(Npow, Npow)
        P = _mm(P, eyeL + Npow)
    U = _mm(P, RHS)

    Pm = G_incl * _mm_t(qc, kc)
    O = A * _mm_t(qc, St) + _mm(Pm, U)
    out_ref[0] = _mm(O, wo[:]) + bo[0]

    lcl = Lc[L - 1, 0]
    gam = jnp.exp(lcl - Lc)
    S[:] = jnp.exp(lcl) * St + _dot(U * gam, kc, ((0,), (0,)))


@jax.jit
def kernel(x, Wq, bq, Wk, bk, Wv, bv, Wa, ba, Wb, bb, Wo, bo):
    B, N, C = x.shape
    grid = (B, N // L)
    wspec = pl.BlockSpec((C, C), lambda b, j: (0, 0))
    bspec = pl.BlockSpec((1, C), lambda b, j: (0, 0))
    xspec = pl.BlockSpec((1, L, C), lambda b, j: (b, j, 0))
    ws = [Wq.T, bq.reshape(1, C), Wk.T, bk.reshape(1, C), Wv.T,
          bv.reshape(1, C), Wa.T, ba.reshape(1, C), Wb.T, bb.reshape(1, C),
          Wo.T, bo.reshape(1, C)]
    out = pl.pallas_call(
        _chunk_kernel,
        grid=grid,
        in_specs=[xspec] + [wspec, bspec] * 6,
        out_specs=xspec,
        out_shape=jax.ShapeDtypeStruct((B, N, C), jnp.float32),
        scratch_shapes=[pltpu.VMEM((C, C), jnp.float32)],
        compiler_params=pltpu.CompilerParams(
            dimension_semantics=("parallel", "arbitrary")),
    )(x, *ws)
    return out


# speedup vs baseline: 27.0809x; 1.0678x over previous
import jax
import jax.numpy as jnp
from jax.experimental import pallas as pl
from jax.experimental.pallas import tpu as pltpu

EPS = 1e-6
L = 128
BB = 2


def _dot(a, b, dims):
    return jax.lax.dot_general(a, b, (dims, ((), ())),
                               preferred_element_type=jnp.float32)


def _mm(a, b):
    return _dot(a, b, ((1,), (0,)))


def _mm_t(a, b):
    return _dot(a, b, ((1,), (1,)))


def _chunk_kernel(x_ref, w5, b5, wo, bo, out_ref, S):
    j = pl.program_id(1)

    @pl.when(j == 0)
    def _():
        S[:] = jnp.zeros_like(S)

    C = w5.shape[0]
    row = jax.lax.broadcasted_iota(jnp.int32, (L, L), 0)
    col = jax.lax.broadcasted_iota(jnp.int32, (L, L), 1)
    tril = (row >= col).astype(jnp.float32)
    eyeL = (row == col).astype(jnp.float32)

    def one(bi):
        xc = x_ref[bi]
        Z = _mm(xc, w5[:]) + b5[0]
        pre_q = Z[:, :C]
        pre_k = Z[:, C:2 * C]
        pre_v = Z[:, 2 * C:3 * C]
        ag = jnp.mean(jax.nn.sigmoid(Z[:, 3 * C:4 * C]), axis=-1,
                      keepdims=True)
        bg = jnp.mean(jax.nn.sigmoid(Z[:, 4 * C:]), axis=-1, keepdims=True)

        def silu(t):
            return t * jax.nn.sigmoid(t)

        def l2n(t):
            return t / (jnp.sqrt(jnp.sum(t * t, axis=-1, keepdims=True)) + EPS)

        qc = l2n(silu(pre_q))
        kc = l2n(silu(pre_k))
        vc = silu(pre_v)

        la = jnp.log(jnp.maximum(ag, 1e-30))
        Lc = _mm(tril, la)
        A = jnp.exp(Lc)
        D = Lc - Lc.reshape(1, L)
        G_strict = jnp.exp(jnp.where(row > col, D, -1e30))
        G_incl = G_strict + eyeL

        St = S[bi]
        KS0 = _mm_t(kc, St)
        RHS = bg * (vc - A * KS0)
        Nm = bg * (G_strict * _mm_t(kc, kc))
        P = eyeL - Nm
        Npow = Nm
        for _ in range(L.bit_length() - 2):
            Npow = _mm(Npow, Npow)
            P = _mm(P, eyeL + Npow)
        U = _mm(P, RHS)

        Pm = G_incl * _mm_t(qc, kc)
        O = A * _mm_t(qc, St) + _mm(Pm, U)
        out_ref[bi] = _mm(O, wo[:]) + bo[0]

        lcl = Lc[L - 1, 0]
        gam = jnp.exp(lcl - Lc)
        S[bi] = jnp.exp(lcl) * St + _dot(U * gam, kc, ((0,), (0,)))

    for bi in range(BB):
        one(bi)


@jax.jit
def kernel(x, Wq, bq, Wk, bk, Wv, bv, Wa, ba, Wb, bb, Wo, bo):
    B, N, C = x.shape
    grid = (B // BB, N // L)
    W5 = jnp.concatenate([Wq.T, Wk.T, Wv.T, Wa.T, Wb.T], axis=1)
    b5 = jnp.concatenate([bq, bk, bv, ba, bb]).reshape(1, 5 * C)
    xspec = pl.BlockSpec((BB, L, C), lambda b, j: (b, j, 0))
    out = pl.pallas_call(
        _chunk_kernel,
        grid=grid,
        in_specs=[xspec,
                  pl.BlockSpec((C, 5 * C), lambda b, j: (0, 0)),
                  pl.BlockSpec((1, 5 * C), lambda b, j: (0, 0)),
                  pl.BlockSpec((C, C), lambda b, j: (0, 0)),
                  pl.BlockSpec((1, C), lambda b, j: (0, 0))],
        out_specs=xspec,
        out_shape=jax.ShapeDtypeStruct((B, N, C), jnp.float32),
        scratch_shapes=[pltpu.VMEM((BB, C, C), jnp.float32)],
        compiler_params=pltpu.CompilerParams(
            dimension_semantics=("parallel", "arbitrary")),
    )(x, W5, b5, Wo.T, bo.reshape(1, C))
    return out
